# baseline (speedup 1.0000x reference)
import functools

import jax
import jax.numpy as jnp
from jax import lax
from jax.experimental import pallas as pl
from jax.experimental.pallas import tpu as pltpu
from jax.experimental.pallas import tpu_sc as plsc

N = 10000
E = 320000
D = 128
DE = 16
L = 3

NC = 2
NS = 16
LANES = 16
NW = NC * NS

B = 64
NEBUF = 3
NHBUF = 2
UNROLL = NEBUF * NHBUF
CH0 = 270
CH1 = 54
SC0TOT = NS * CH0 * B
EP = NS * (CH0 + CH1) * B
PADE = EP - E
NTRASH = 112
NPAD = N + NTRASH
RPT = NPAD // NS

_sc_mesh = plsc.VectorSubcoreMesh(
    core_axis_name="c", subcore_axis_name="s", num_cores=NC, num_subcores=NS)


@functools.partial(
    pl.kernel,
    out_type=jax.ShapeDtypeStruct((NC, NPAD, D), jnp.float32),
    mesh=_sc_mesh,
    scratch_types=[
        pltpu.VMEM((NEBUF, B), jnp.int32),
        pltpu.VMEM((NEBUF, B), jnp.int32),
        pltpu.VMEM((NHBUF, B, D), jnp.float32),
        pltpu.VMEM((NEBUF, B, D), jnp.float32),
        pltpu.VMEM_SHARED((NPAD, D), jnp.float32),
        pltpu.SemaphoreType.DMA((NEBUF,)),
        pltpu.SemaphoreType.DMA((NHBUF,)),
        pltpu.SemaphoreType.DMA((NHBUF,)),
    ],
    compiler_params=pltpu.CompilerParams(needs_layout_passes=False),
)
def _edge_stage(h_hbm, e_hbm, src_hbm, dst_hbm, out_hbm,
                srcv, dstv, hbuf, ebuf, agg_sh, sem_in, sem_g, sem_s):
    c = lax.axis_index("c")
    s = lax.axis_index("s")
    r0 = s * RPT
    wbase = jnp.where(c == 0, s * (CH0 * B), SC0TOT + s * (CH1 * B))

    def _in_copies(i, be):
        base = wbase + i * B
        return ((src_hbm.at[pl.ds(base, B)], srcv.at[be]),
                (dst_hbm.at[pl.ds(base, B)], dstv.at[be]),
                (e_hbm.at[pl.ds(base, B)], ebuf.at[be]))

    def _start_inputs(i, be):
        for s_, d_ in _in_copies(i, be):
            pltpu.async_copy(s_, d_, sem_in.at[be])

    def _wait_inputs(i, be):
        for s_, d_ in _in_copies(i, be):
            pltpu.make_async_copy(s_, d_, sem_in.at[be]).wait()

    def _start_gather(be, bh):
        pltpu.async_copy(h_hbm.at[srcv.at[be]], hbuf.at[bh], sem_g.at[bh])

    def _wait_gather(be, bh):
        pltpu.make_async_copy(h_hbm.at[srcv.at[be]], hbuf.at[bh],
                              sem_g.at[bh]).wait()

    def _start_scatter(be, bm):
        pltpu.async_copy(hbuf.at[bm], agg_sh.at[dstv.at[be]], sem_s.at[bm],
                         add=True)

    def _wait_scatter(be, bm):
        pltpu.make_async_copy(hbuf.at[bm], agg_sh.at[dstv.at[be]],
                              sem_s.at[bm]).wait()

    def _compute(be, bh):
        @plsc.parallel_loop(0, B, 1, unroll=2)
        def _row(r):
            for cc in range(D // LANES):
                sl = pl.ds(cc * LANES, LANES)
                hbuf[bh, r, sl] = jnp.maximum(hbuf[bh, r, sl] + ebuf[be, r, sl],
                                              0.0)

    @plsc.parallel_loop(0, B, 1, unroll=2)
    def _zrow(r):
        for cc in range(D // LANES):
            hbuf[0, r, pl.ds(cc * LANES, LANES)] = jnp.zeros((LANES,),
                                                             jnp.float32)
    zrem = RPT % B
    zcps = [(hbuf.at[0], agg_sh.at[pl.ds(r0 + k * B, B)])
            for k in range(RPT // B)]
    zcps.append((hbuf.at[0].at[pl.ds(0, zrem)],
                 agg_sh.at[pl.ds(r0 + (RPT // B) * B, zrem)]))
    for s_, d_ in zcps:
        pltpu.async_copy(s_, d_, sem_g.at[0])
    for s_, d_ in zcps:
        pltpu.make_async_copy(s_, d_, sem_g.at[0]).wait()
    plsc.subcore_barrier()

    _start_inputs(0, 0)
    _start_inputs(1, 1)
    _wait_inputs(0, 0)
    _start_gather(0, 0)

    def _make_group(chunks):
        def _group(j, carry):
            for b in range(UNROLL):
                i = j * UNROLL + b
                be = b % NEBUF
                bh = b % NHBUF
                bs = (b + 2) % NEBUF
                bg = (b + 1) % NEBUF
                bgh = (b + 1) % NHBUF

                @pl.when(i >= 1)
                def _():
                    _wait_scatter(bs, bgh)

                @pl.when(i + 2 < chunks)
                def _():
                    _start_inputs(j * UNROLL + b + 2, bs)

                @pl.when(i + 1 < chunks)
                def _():
                    _wait_inputs(j * UNROLL + b + 1, bg)
                    _start_gather(bg, bgh)

                _wait_gather(be, bh)
                _compute(be, bh)
                _start_scatter(be, bh)
            return carry
        return _group

    @pl.when(c == 0)
    def _():
        lax.fori_loop(0, CH0 // UNROLL, _make_group(CH0), 0)

    @pl.when(c == 1)
    def _():
        lax.fori_loop(0, CH1 // UNROLL, _make_group(CH1), 0)

    _wait_scatter(2, 1)
    plsc.subcore_barrier()

    obufs = [hbuf.at[0], hbuf.at[1]]
    nout = RPT // B + 1
    orem = RPT % B
    for rnd in range((nout + 1) // 2):
        cps = []
        for k in range(rnd * 2, min((rnd + 1) * 2, nout)):
            nrows = B if k < RPT // B else orem
            bb = obufs[k % 2] if nrows == B else obufs[k % 2].at[pl.ds(0, nrows)]
            cps.append((agg_sh.at[pl.ds(r0 + k * B, nrows)], bb,
                        out_hbm.at[c, pl.ds(r0 + k * B, nrows)]))
        for src_, buf_, dst_ in cps:
            pltpu.sync_copy(src_, buf_)
            pltpu.async_copy(buf_, dst_, sem_s.at[0])
        for src_, buf_, dst_ in cps:
            pltpu.make_async_copy(buf_, dst_, sem_s.at[0]).wait()


def _enc_body(x_ref, w_ref, b_ref, o_ref):
    o_ref[...] = jnp.dot(x_ref[...], w_ref[...],
                         preferred_element_type=jnp.float32) + b_ref[...]


BE = 1024


def _eenc_body(ea_ref, we_ref, be_ref, o_ref):
    rows = pl.program_id(0) * BE + lax.broadcasted_iota(jnp.int32, (BE, 1), 0)
    v = jnp.dot(ea_ref[...], we_ref[...],
                preferred_element_type=jnp.float32) + be_ref[...]
    o_ref[...] = jnp.where(rows < E, v, -1e30)


def _node_body(h_ref, a_ref, eps_ref, w1_ref, b1_ref, g1_ref, bb1_ref,
               w2_ref, b2_ref, g2_ref, bb2_ref, o_ref, *, last):
    agg = a_ref[0, :N, :] + a_ref[1, :N, :]
    z = (1.0 + eps_ref[0, 0]) * h_ref[...] + agg
    z = jnp.dot(z, w1_ref[...], preferred_element_type=jnp.float32) + b1_ref[...]
    m = jnp.mean(z, axis=0, keepdims=True)
    v = jnp.mean((z - m) ** 2, axis=0, keepdims=True)
    z = g1_ref[...] * (z - m) * lax.rsqrt(v + 1e-5) + bb1_ref[...]
    z = jnp.maximum(z, 0.0)
    z = jnp.dot(z, w2_ref[...], preferred_element_type=jnp.float32) + b2_ref[...]
    m2 = jnp.mean(z, axis=0, keepdims=True)
    v2 = jnp.mean((z - m2) ** 2, axis=0, keepdims=True)
    z = g2_ref[...] * (z - m2) * lax.rsqrt(v2 + 1e-5) + bb2_ref[...]
    if not last:
        z = jnp.maximum(z, 0.0)
    o_ref[...] = z


def kernel(x, edge_index, edge_attr, batch, W_enc, b_enc, eps, We, bee,
           W1, b1, g1, bb1, W2, b2, g2, bb2):
    src = jnp.concatenate([edge_index[0], jnp.zeros((PADE,), jnp.int32)])
    dst = jnp.concatenate(
        [edge_index[1], jnp.arange(PADE, dtype=jnp.int32) % N])
    ea = jnp.concatenate([edge_attr, jnp.zeros((PADE, DE), jnp.float32)], axis=0)

    h = pl.pallas_call(
        _enc_body,
        out_shape=jax.ShapeDtypeStruct((N, D), jnp.float32),
    )(x, W_enc, b_enc.reshape(1, D))

    def _eenc(l):
        return pl.pallas_call(
            _eenc_body,
            grid=(EP // BE,),
            in_specs=[
                pl.BlockSpec((BE, DE), lambda i: (i, 0)),
                pl.BlockSpec((DE, D), lambda i: (0, 0)),
                pl.BlockSpec((1, D), lambda i: (0, 0)),
            ],
            out_specs=pl.BlockSpec((BE, D), lambda i: (i, 0)),
            out_shape=jax.ShapeDtypeStruct((EP, D), jnp.float32),
        )(ea, We[l], bee[l].reshape(1, D))

    for l in range(L):
        agg2 = _edge_stage(h, _eenc(l), src, dst)
        h = pl.pallas_call(
            functools.partial(_node_body, last=(l == L - 1)),
            out_shape=jax.ShapeDtypeStruct((N, D), jnp.float32),
        )(h, agg2, eps[l].reshape(1, 1),
          W1[l], b1[l].reshape(1, 2 * D), g1[l].reshape(1, 2 * D),
          bb1[l].reshape(1, 2 * D),
          W2[l], b2[l].reshape(1, D), g2[l].reshape(1, D),
          bb2[l].reshape(1, D))
    return h


# speedup vs baseline: 1.0135x; 1.0135x over previous
import functools

import jax
import jax.numpy as jnp
from jax import lax
from jax.experimental import pallas as pl
from jax.experimental.pallas import tpu as pltpu
from jax.experimental.pallas import tpu_sc as plsc

N = 10000
E = 320000
D = 128
DE = 16
L = 3

NC = 2
NS = 16
LANES = 16
NW = NC * NS

B = 64
NEBUF = 3
NHBUF = 2
UNROLL = NEBUF * NHBUF
CH0 = 300
CH1 = 24
SC0TOT = NS * CH0 * B
EP = NS * (CH0 + CH1) * B
PADE = EP - E
NTRASH = 112
NPAD = N + NTRASH
RPT = NPAD // NS

_sc_mesh = plsc.VectorSubcoreMesh(
    core_axis_name="c", subcore_axis_name="s", num_cores=NC, num_subcores=NS)


@functools.partial(
    pl.kernel,
    out_type=jax.ShapeDtypeStruct((NC, NPAD, D), jnp.float32),
    mesh=_sc_mesh,
    scratch_types=[
        pltpu.VMEM((NEBUF, B), jnp.int32),
        pltpu.VMEM((NEBUF, B), jnp.int32),
        pltpu.VMEM((NHBUF, B, D), jnp.float32),
        pltpu.VMEM((NEBUF, B, D), jnp.float32),
        pltpu.VMEM_SHARED((NPAD, D), jnp.float32),
        pltpu.SemaphoreType.DMA((NEBUF,)),
        pltpu.SemaphoreType.DMA((NHBUF,)),
        pltpu.SemaphoreType.DMA((NHBUF,)),
    ],
    compiler_params=pltpu.CompilerParams(needs_layout_passes=False),
)
def _edge_stage(h_hbm, e_hbm, src_hbm, dst_hbm, out_hbm,
                srcv, dstv, hbuf, ebuf, agg_sh, sem_in, sem_g, sem_s):
    c = lax.axis_index("c")
    s = lax.axis_index("s")
    r0 = s * RPT
    wbase = jnp.where(c == 0, s * (CH0 * B), SC0TOT + s * (CH1 * B))

    def _in_copies(i, be):
        base = wbase + i * B
        return ((src_hbm.at[pl.ds(base, B)], srcv.at[be]),
                (dst_hbm.at[pl.ds(base, B)], dstv.at[be]),
                (e_hbm.at[pl.ds(base, B)], ebuf.at[be]))

    def _start_inputs(i, be):
        for s_, d_ in _in_copies(i, be):
            pltpu.async_copy(s_, d_, sem_in.at[be])

    def _wait_inputs(i, be):
        for s_, d_ in _in_copies(i, be):
            pltpu.make_async_copy(s_, d_, sem_in.at[be]).wait()

    def _start_gather(be, bh):
        pltpu.async_copy(h_hbm.at[srcv.at[be]], hbuf.at[bh], sem_g.at[bh])

    def _wait_gather(be, bh):
        pltpu.make_async_copy(h_hbm.at[srcv.at[be]], hbuf.at[bh],
                              sem_g.at[bh]).wait()

    def _start_scatter(be, bm):
        pltpu.async_copy(hbuf.at[bm], agg_sh.at[dstv.at[be]], sem_s.at[bm],
                         add=True)

    def _wait_scatter(be, bm):
        pltpu.make_async_copy(hbuf.at[bm], agg_sh.at[dstv.at[be]],
                              sem_s.at[bm]).wait()

    def _compute(be, bh):
        @plsc.parallel_loop(0, B, 1, unroll=2)
        def _row(r):
            for cc in range(D // LANES):
                sl = pl.ds(cc * LANES, LANES)
                hbuf[bh, r, sl] = jnp.maximum(hbuf[bh, r, sl] + ebuf[be, r, sl],
                                              0.0)

    @plsc.parallel_loop(0, B, 1, unroll=2)
    def _zrow(r):
        for cc in range(D // LANES):
            hbuf[0, r, pl.ds(cc * LANES, LANES)] = jnp.zeros((LANES,),
                                                             jnp.float32)
    zrem = RPT % B
    zcps = [(hbuf.at[0], agg_sh.at[pl.ds(r0 + k * B, B)])
            for k in range(RPT // B)]
    zcps.append((hbuf.at[0].at[pl.ds(0, zrem)],
                 agg_sh.at[pl.ds(r0 + (RPT // B) * B, zrem)]))
    for s_, d_ in zcps:
        pltpu.async_copy(s_, d_, sem_g.at[0])
    for s_, d_ in zcps:
        pltpu.make_async_copy(s_, d_, sem_g.at[0]).wait()
    plsc.subcore_barrier()

    _start_inputs(0, 0)
    _start_inputs(1, 1)
    _wait_inputs(0, 0)
    _start_gather(0, 0)

    def _make_group(chunks):
        def _group(j, carry):
            for b in range(UNROLL):
                i = j * UNROLL + b
                be = b % NEBUF
                bh = b % NHBUF
                bs = (b + 2) % NEBUF
                bg = (b + 1) % NEBUF
                bgh = (b + 1) % NHBUF

                @pl.when(i >= 1)
                def _():
                    _wait_scatter(bs, bgh)

                @pl.when(i + 2 < chunks)
                def _():
                    _start_inputs(j * UNROLL + b + 2, bs)

                @pl.when(i + 1 < chunks)
                def _():
                    _wait_inputs(j * UNROLL + b + 1, bg)
                    _start_gather(bg, bgh)

                _wait_gather(be, bh)
                _compute(be, bh)
                _start_scatter(be, bh)
            return carry
        return _group

    @pl.when(c == 0)
    def _():
        lax.fori_loop(0, CH0 // UNROLL, _make_group(CH0), 0)

    @pl.when(c == 1)
    def _():
        lax.fori_loop(0, CH1 // UNROLL, _make_group(CH1), 0)

    _wait_scatter(2, 1)
    plsc.subcore_barrier()

    obufs = [hbuf.at[0], hbuf.at[1]]
    nout = RPT // B + 1
    orem = RPT % B
    for rnd in range((nout + 1) // 2):
        cps = []
        for k in range(rnd * 2, min((rnd + 1) * 2, nout)):
            nrows = B if k < RPT // B else orem
            bb = obufs[k % 2] if nrows == B else obufs[k % 2].at[pl.ds(0, nrows)]
            cps.append((agg_sh.at[pl.ds(r0 + k * B, nrows)], bb,
                        out_hbm.at[c, pl.ds(r0 + k * B, nrows)]))
        for src_, buf_, dst_ in cps:
            pltpu.sync_copy(src_, buf_)
            pltpu.async_copy(buf_, dst_, sem_s.at[0])
        for src_, buf_, dst_ in cps:
            pltpu.make_async_copy(buf_, dst_, sem_s.at[0]).wait()


def _enc_body(x_ref, w_ref, b_ref, o_ref):
    o_ref[...] = jnp.dot(x_ref[...], w_ref[...],
                         preferred_element_type=jnp.float32) + b_ref[...]


BE = 1024


def _eenc_body(ea_ref, we_ref, be_ref, o_ref):
    rows = pl.program_id(0) * BE + lax.broadcasted_iota(jnp.int32, (BE, 1), 0)
    v = jnp.dot(ea_ref[...], we_ref[...],
                preferred_element_type=jnp.float32) + be_ref[...]
    o_ref[...] = jnp.where(rows < E, v, -1e30)


def _node_body(h_ref, a_ref, eps_ref, w1_ref, b1_ref, g1_ref, bb1_ref,
               w2_ref, b2_ref, g2_ref, bb2_ref, o_ref, *, last):
    agg = a_ref[0, :N, :] + a_ref[1, :N, :]
    z = (1.0 + eps_ref[0, 0]) * h_ref[...] + agg
    z = jnp.dot(z, w1_ref[...], preferred_element_type=jnp.float32) + b1_ref[...]
    m = jnp.mean(z, axis=0, keepdims=True)
    v = jnp.mean((z - m) ** 2, axis=0, keepdims=True)
    z = g1_ref[...] * (z - m) * lax.rsqrt(v + 1e-5) + bb1_ref[...]
    z = jnp.maximum(z, 0.0)
    z = jnp.dot(z, w2_ref[...], preferred_element_type=jnp.float32) + b2_ref[...]
    m2 = jnp.mean(z, axis=0, keepdims=True)
    v2 = jnp.mean((z - m2) ** 2, axis=0, keepdims=True)
    z = g2_ref[...] * (z - m2) * lax.rsqrt(v2 + 1e-5) + bb2_ref[...]
    if not last:
        z = jnp.maximum(z, 0.0)
    o_ref[...] = z


def kernel(x, edge_index, edge_attr, batch, W_enc, b_enc, eps, We, bee,
           W1, b1, g1, bb1, W2, b2, g2, bb2):
    src = jnp.concatenate([edge_index[0], jnp.zeros((PADE,), jnp.int32)])
    dst = jnp.concatenate(
        [edge_index[1], jnp.arange(PADE, dtype=jnp.int32) % N])
    ea = jnp.concatenate([edge_attr, jnp.zeros((PADE, DE), jnp.float32)], axis=0)

    h = pl.pallas_call(
        _enc_body,
        out_shape=jax.ShapeDtypeStruct((N, D), jnp.float32),
    )(x, W_enc, b_enc.reshape(1, D))

    def _eenc(l):
        return pl.pallas_call(
            _eenc_body,
            grid=(EP // BE,),
            in_specs=[
                pl.BlockSpec((BE, DE), lambda i: (i, 0)),
                pl.BlockSpec((DE, D), lambda i: (0, 0)),
                pl.BlockSpec((1, D), lambda i: (0, 0)),
            ],
            out_specs=pl.BlockSpec((BE, D), lambda i: (i, 0)),
            out_shape=jax.ShapeDtypeStruct((EP, D), jnp.float32),
        )(ea, We[l], bee[l].reshape(1, D))

    for l in range(L):
        agg2 = _edge_stage(h, _eenc(l), src, dst)
        h = pl.pallas_call(
            functools.partial(_node_body, last=(l == L - 1)),
            out_shape=jax.ShapeDtypeStruct((N, D), jnp.float32),
        )(h, agg2, eps[l].reshape(1, 1),
          W1[l], b1[l].reshape(1, 2 * D), g1[l].reshape(1, 2 * D),
          bb1[l].reshape(1, 2 * D),
          W2[l], b2[l].reshape(1, D), g2[l].reshape(1, D),
          bb2[l].reshape(1, D))
    return h
